# baseline (speedup 1.0000x reference)
import functools

import jax
import jax.numpy as jnp
from jax import lax
from jax.experimental import pallas as pl
from jax.experimental.pallas import tpu as pltpu
from jax.experimental.pallas import tpu_sc as plsc

_FAN_IN = 1024
_FAN_OUT = 1024
_K = 2048
_NW = 32
_NJ = _FAN_IN + 1
_NCB = _FAN_OUT // 128
_JPW = 32
_PER_W = _JPW * _FAN_OUT
_JG = 1032
_G_TOTAL = _NCB * _JG * 128


def _gather_body(w_hbm, h_hbm, g_hbm, w_v, h_v, g_v, hx_v, gx_v):
    wid = lax.axis_index("s") * 2 + lax.axis_index("c")
    pltpu.sync_copy(w_hbm, w_v)
    pltpu.sync_copy(h_hbm.at[pl.ds(_PER_W * wid, _PER_W)], h_v)

    @plsc.parallel_loop(0, _JPW, 1, unroll=2)
    def _(x):
        hbase = x * 1024
        gbase = x * 128
        for cb in range(_NCB):
            for v in range(8):
                idx = h_v[pl.ds(hbase + cb * 128 + 16 * v, 16)]
                val = plsc.load_gather(w_v, [idx])
                g_v[pl.ds(cb * _JPW * 128 + gbase + 16 * v, 16)] = val

    for cb in range(_NCB):
        blk = _JPW * 128
        pltpu.sync_copy(
            g_v.at[pl.ds(cb * blk, blk)],
            g_hbm.at[pl.ds((cb * _JG + _JPW * wid) * 128, blk)],
        )

    @pl.when(wid < _NCB)
    def _():
        pltpu.sync_copy(h_hbm.at[pl.ds(1024 * 1024 + 128 * wid, 128)], hx_v)
        for v in range(8):
            idx = hx_v[pl.ds(16 * v, 16)]
            val = plsc.load_gather(w_v, [idx])
            gx_v[pl.ds(16 * v, 16)] = val
        zero = jnp.zeros((16,), jnp.float32)
        for z in range(8, 64):
            gx_v[pl.ds(16 * z, 16)] = zero
        pltpu.sync_copy(
            gx_v, g_hbm.at[pl.ds((wid * _JG + 1024) * 128, 1024)])


_gather = functools.partial(
    pl.kernel,
    mesh=plsc.VectorSubcoreMesh(core_axis_name="c", subcore_axis_name="s"),
    out_type=jax.ShapeDtypeStruct((_G_TOTAL,), jnp.float32),
    scratch_types=[
        pltpu.VMEM((_K,), jnp.float32),
        pltpu.VMEM((_PER_W,), jnp.int32),
        pltpu.VMEM((_JPW * 1024,), jnp.float32),
        pltpu.VMEM((128,), jnp.int32),
        pltpu.VMEM((1024,), jnp.float32),
    ],
    compiler_params=pltpu.CompilerParams(needs_layout_passes=False),
)(_gather_body)


def _matmul_body(a_ref, g_ref, o_ref):
    a = a_ref[...]
    b = a.shape[0]
    tail = (lax.broadcasted_iota(jnp.int32, (b, _JG - _FAN_IN), 1) == 0)
    a_aug = jnp.concatenate([a, tail.astype(jnp.float32)], axis=1)
    o_ref[...] = lax.dot_general(
        a_aug,
        g_ref[...],
        (((1,), (0,)), ((), ())),
        preferred_element_type=jnp.float32,
    )


def kernel(a, W, H):
    g = _gather(W, H.T.reshape(-1))
    g2 = g.reshape(_NCB * _JG, 128)
    b = a.shape[0]
    return pl.pallas_call(
        _matmul_body,
        grid=(_NCB,),
        in_specs=[
            pl.BlockSpec((b, _FAN_IN), lambda c: (0, 0)),
            pl.BlockSpec((_JG, 128), lambda c: (c, 0)),
        ],
        out_specs=pl.BlockSpec((b, 128), lambda c: (0, c)),
        out_shape=jax.ShapeDtypeStruct((b, _FAN_OUT), jnp.float32),
    )(a, g2)


# speedup vs baseline: 1.0820x; 1.0820x over previous
import functools

import jax
import jax.numpy as jnp
from jax import lax
from jax.experimental import pallas as pl
from jax.experimental.pallas import tpu as pltpu
from jax.experimental.pallas import tpu_sc as plsc

_FAN_IN = 1024
_FAN_OUT = 1024
_K = 2048
_NW = 32
_NJ = _FAN_IN + 1
_NCB = _FAN_OUT // 128
_JPW = 32
_PER_W = _JPW * _FAN_OUT
_JG = 1032
_G_TOTAL = _NCB * _JG * 128


def _gather_body(w_hbm, h_hbm, g_hbm, w_v, h_v, g_v, hx_v, gx_v):
    wid = lax.axis_index("s") * 2 + lax.axis_index("c")
    pltpu.sync_copy(w_hbm, w_v)
    pltpu.sync_copy(h_hbm.at[pl.ds(_PER_W * wid, _PER_W)], h_v)

    @plsc.parallel_loop(0, _JPW, 1, unroll=2)
    def _(x):
        hbase = x * 1024
        gbase = x * 128
        for cb in range(_NCB):
            for v in range(8):
                idx = h_v[pl.ds(hbase + cb * 128 + 16 * v, 16)]
                val = plsc.load_gather(w_v, [idx])
                g_v[pl.ds(cb * _JPW * 128 + gbase + 16 * v, 16)] = val

    for cb in range(_NCB):
        blk = _JPW * 128
        pltpu.sync_copy(
            g_v.at[pl.ds(cb * blk, blk)],
            g_hbm.at[pl.ds((cb * _JG + _JPW * wid) * 128, blk)],
        )

    @pl.when(wid < _NCB)
    def _():
        pltpu.sync_copy(h_hbm.at[pl.ds(1024 * 1024 + 128 * wid, 128)], hx_v)
        for v in range(8):
            idx = hx_v[pl.ds(16 * v, 16)]
            val = plsc.load_gather(w_v, [idx])
            gx_v[pl.ds(16 * v, 16)] = val
        zero = jnp.zeros((16,), jnp.float32)
        for z in range(8, 64):
            gx_v[pl.ds(16 * z, 16)] = zero
        pltpu.sync_copy(
            gx_v, g_hbm.at[pl.ds((wid * _JG + 1024) * 128, 1024)])


_gather = functools.partial(
    pl.kernel,
    mesh=plsc.VectorSubcoreMesh(core_axis_name="c", subcore_axis_name="s"),
    out_type=jax.ShapeDtypeStruct((_G_TOTAL,), jnp.float32),
    scratch_types=[
        pltpu.VMEM((_K,), jnp.float32),
        pltpu.VMEM((_PER_W,), jnp.int32),
        pltpu.VMEM((_JPW * 1024,), jnp.float32),
        pltpu.VMEM((128,), jnp.int32),
        pltpu.VMEM((1024,), jnp.float32),
    ],
    compiler_params=pltpu.CompilerParams(needs_layout_passes=False),
)(_gather_body)


def _matmul_body(a_ref, g_ref, o_ref):
    a = a_ref[...]
    b = a.shape[0]
    tail = (lax.broadcasted_iota(jnp.int32, (b, _JG - _FAN_IN), 1) == 0)
    a_aug = jnp.concatenate([a, tail.astype(jnp.float32)], axis=1)
    for c in range(_NCB):
        o_ref[:, c * 128:(c + 1) * 128] = lax.dot_general(
            a_aug,
            g_ref[pl.ds(c * _JG, _JG), :],
            (((1,), (0,)), ((), ())),
            preferred_element_type=jnp.float32,
        )


def kernel(a, W, H):
    g = _gather(W, H.T.reshape(-1))
    g2 = g.reshape(_NCB * _JG, 128)
    return pl.pallas_call(
        _matmul_body,
        out_shape=jax.ShapeDtypeStruct((a.shape[0], _FAN_OUT), jnp.float32),
    )(a, g2)
